# baseline (speedup 1.0000x reference)
import jax
import jax.numpy as jnp
from jax.experimental import pallas as pl
from jax.experimental.pallas import tpu as pltpu

_CONF = 0.01
_S = 4
_BLK = 1000


def _infbox_body(*refs):
    preds = refs[:_S]
    dbox_ref = refs[_S]
    loc_ref, ind_ref = refs[_S + 1:]
    for k in range(_S):
        pred = preds[k][0]
        rows = pl.ds(k * _BLK, _BLK)
        d = dbox_ref[rows, :]
        ind_ref[0, rows, :] = pred[:, 4:] > _CONF
        p = pred[:, :4]
        ctr = d[:, :2] + 0.1 * p[:, :2] * d[:, 2:]
        half = 0.5 * d[:, 2:] * jnp.exp(0.2 * p[:, 2:])
        loc_ref[0, rows, :] = jnp.concatenate([ctr - half, ctr + half], axis=1)


def kernel(predicts, dboxes):
    batch, n, c = predicts.shape
    step_rows = _S * _BLK
    nblk = n // step_rows

    pred_specs = [
        pl.BlockSpec((1, _BLK, c), lambda b, j, k=k: (b, j * _S + k, 0))
        for k in range(_S)
    ]
    loc, ind = pl.pallas_call(
        _infbox_body,
        grid=(batch, nblk),
        in_specs=pred_specs
        + [pl.BlockSpec((step_rows, 4), lambda b, j: (j, 0))],
        out_specs=[
            pl.BlockSpec((1, step_rows, 4), lambda b, j: (b, j, 0)),
            pl.BlockSpec((1, step_rows, c - 4), lambda b, j: (b, j, 0)),
        ],
        out_shape=[
            jax.ShapeDtypeStruct((batch, n, 4), jnp.float32),
            jax.ShapeDtypeStruct((batch, n, c - 4), jnp.bool_),
        ],
        compiler_params=pltpu.CompilerParams(
            dimension_semantics=("parallel", "parallel"),
        ),
    )(*([predicts] * _S + [dboxes]))
    return (loc, ind)
